# baseline (speedup 1.0000x reference)
import jax
import jax.numpy as jnp
from jax.experimental import pallas as pl
from jax.experimental.pallas import tpu as pltpu

NUM_LABELS = 1000
MEM_PER_LABEL = 128
MODEL_DIM = 512
ROWS = NUM_LABELS * MEM_PER_LABEL
BR = 2000
NSTEPS = ROWS // BR
NBUF = 2


def _step_compute(ibuf, mbuf, fbuf, obuf, b):
    valid = mbuf[b] * fbuf[b]
    obuf[b] = ibuf[b] * valid


def _start_in(flat, mask, flag, ibuf, mbuf, fbuf, isem, msem, fsem, s, b):
    rows = pl.ds(s * BR, BR)
    pltpu.make_async_copy(flat.at[rows], ibuf.at[b], isem.at[b]).start()
    pltpu.make_async_copy(mask.at[rows], mbuf.at[b], msem.at[b]).start()
    pltpu.make_async_copy(flag.at[rows], fbuf.at[b], fsem.at[b]).start()


def _wait_in(flat, mask, flag, ibuf, mbuf, fbuf, isem, msem, fsem, s, b):
    rows = pl.ds(s * BR, BR)
    pltpu.make_async_copy(flat.at[rows], ibuf.at[b], isem.at[b]).wait()
    pltpu.make_async_copy(mask.at[rows], mbuf.at[b], msem.at[b]).wait()
    pltpu.make_async_copy(flag.at[rows], fbuf.at[b], fsem.at[b]).wait()


def _masked_copy_kernel(flat, mask, flag, out,
                        ibuf, mbuf, fbuf, obuf, isem, msem, fsem, osem):
    def start_in(s, b):
        _start_in(flat, mask, flag, ibuf, mbuf, fbuf, isem, msem, fsem, s, b)

    def wait_in(s, b):
        _wait_in(flat, mask, flag, ibuf, mbuf, fbuf, isem, msem, fsem, s, b)

    for b in range(NBUF):
        start_in(b, b)
    for s in range(NBUF):
        b = s % NBUF
        wait_in(s, b)
        _step_compute(ibuf, mbuf, fbuf, obuf, b)
        pltpu.make_async_copy(obuf.at[b], out.at[pl.ds(s * BR, BR)], osem.at[b]).start()
        start_in(s + NBUF, b)

    def body(s2, _):
        for b in range(NBUF):
            s = s2 * NBUF + b
            wait_in(s, b)
            pltpu.make_async_copy(
                obuf.at[b], out.at[pl.ds((s - NBUF) * BR, BR)], osem.at[b]
            ).wait()
            _step_compute(ibuf, mbuf, fbuf, obuf, b)
            pltpu.make_async_copy(
                obuf.at[b], out.at[pl.ds(s * BR, BR)], osem.at[b]
            ).start()
            start_in(s + NBUF, b)
        return 0

    jax.lax.fori_loop(1, NSTEPS // NBUF - 1, body, 0)

    for s in range(NSTEPS - NBUF, NSTEPS):
        b = s % NBUF
        wait_in(s, b)
        pltpu.make_async_copy(
            obuf.at[b], out.at[pl.ds((s - NBUF) * BR, BR)], osem.at[b]
        ).wait()
        _step_compute(ibuf, mbuf, fbuf, obuf, b)
        pltpu.make_async_copy(
            obuf.at[b], out.at[pl.ds(s * BR, BR)], osem.at[b]
        ).start()
    for s in range(NSTEPS - NBUF, NSTEPS):
        b = s % NBUF
        pltpu.make_async_copy(
            obuf.at[b], out.at[pl.ds(s * BR, BR)], osem.at[b]
        ).wait()


def kernel(memory, memory_mask, memory_collected_flag):
    flat = memory.reshape(ROWS, MODEL_DIM)
    mask_col = memory_mask.reshape(ROWS, 1).astype(jnp.float32)
    flag_col = (
        jnp.broadcast_to(memory_collected_flag[:, None], (NUM_LABELS, MEM_PER_LABEL))
        .reshape(ROWS, 1)
        .astype(jnp.float32)
    )
    return pl.pallas_call(
        _masked_copy_kernel,
        in_specs=[
            pl.BlockSpec(memory_space=pl.ANY),
            pl.BlockSpec(memory_space=pl.ANY),
            pl.BlockSpec(memory_space=pl.ANY),
        ],
        out_specs=pl.BlockSpec(memory_space=pl.ANY),
        out_shape=jax.ShapeDtypeStruct((ROWS, MODEL_DIM), jnp.float32),
        scratch_shapes=[
            pltpu.VMEM((NBUF, BR, MODEL_DIM), jnp.float32),
            pltpu.VMEM((NBUF, BR, 1), jnp.float32),
            pltpu.VMEM((NBUF, BR, 1), jnp.float32),
            pltpu.VMEM((NBUF, BR, MODEL_DIM), jnp.float32),
            pltpu.SemaphoreType.DMA((NBUF,)),
            pltpu.SemaphoreType.DMA((NBUF,)),
            pltpu.SemaphoreType.DMA((NBUF,)),
            pltpu.SemaphoreType.DMA((NBUF,)),
        ],
    )(flat, mask_col, flag_col)


# speedup vs baseline: 1.2456x; 1.2456x over previous
import functools
import jax
import jax.numpy as jnp
from jax import lax
from jax.experimental import pallas as pl
from jax.experimental.pallas import tpu as pltpu
from jax.experimental.pallas import tpu_sc as plsc

NUM_LABELS = 1000
MEM_PER_LABEL = 128
MODEL_DIM = 512
ROWS = NUM_LABELS * MEM_PER_LABEL
NC, NS, L = 2, 16, 16
NW = NC * NS
RPW = ROWS // NW
CHUNK = 80
N_CHUNKS = RPW // CHUNK
NBUF = 2


def _sc_body(flat_hbm, mask_hbm, flag_hbm, out_hbm,
             buf, mask_v, flag_v, isem, osem, psem):
    wid = lax.axis_index("s") * NC + lax.axis_index("c")
    base = wid * RPW
    mcp = pltpu.make_async_copy(mask_hbm.at[pl.ds(base, RPW)], mask_v, psem)
    fcp = pltpu.make_async_copy(flag_hbm.at[pl.ds(base, RPW)], flag_v, psem)
    mcp.start()
    fcp.start()
    mcp.wait()
    fcp.wait()

    def start_in(i, b):
        cbase = pl.multiple_of(base + i * CHUNK, 8)
        pltpu.make_async_copy(
            flat_hbm.at[pl.ds(cbase, CHUNK)], buf.at[b], isem.at[b]
        ).start()

    def wait_in(i, b):
        cbase = pl.multiple_of(base + i * CHUNK, 8)
        pltpu.make_async_copy(
            flat_hbm.at[pl.ds(cbase, CHUNK)], buf.at[b], isem.at[b]
        ).wait()

    def out_dma(i, b):
        cbase = pl.multiple_of(base + i * CHUNK, 8)
        return pltpu.make_async_copy(
            buf.at[b], out_hbm.at[pl.ds(cbase, CHUNK)], osem.at[b]
        )

    def compute(i, b):
        def group_body(g, _):
            off = pl.multiple_of(i * CHUNK + g * L, 8)
            mv = mask_v[pl.ds(off, L)] * flag_v[pl.ds(off, L)]
            for r in range(L):
                sval = mv[r]
                for j in range(MODEL_DIM // L):
                    sl = pl.ds(j * L, L)
                    buf[b, g * L + r, sl] = buf[b, g * L + r, sl] * sval
            return 0

        lax.fori_loop(0, CHUNK // L, group_body, 0)

    for b in range(NBUF):
        start_in(b, b)
    for i in range(NBUF):
        b = i % NBUF
        wait_in(i, b)
        compute(i, b)
        out_dma(i, b).start()
        start_in(i + NBUF, b)

    def loop_body(k, _):
        for b in range(NBUF):
            i = k * NBUF + b
            wait_in(i, b)
            out_dma(i - NBUF, b).wait()
            compute(i, b)
            out_dma(i, b).start()
            start_in(i + NBUF, b)
        return 0

    lax.fori_loop(1, N_CHUNKS // NBUF - 1, loop_body, 0)

    for i in range(N_CHUNKS - NBUF, N_CHUNKS):
        b = i % NBUF
        wait_in(i, b)
        out_dma(i - NBUF, b).wait()
        compute(i, b)
        out_dma(i, b).start()
    for i in range(N_CHUNKS - NBUF, N_CHUNKS):
        out_dma(i, i % NBUF).wait()


def kernel(memory, memory_mask, memory_collected_flag):
    flat = memory.reshape(ROWS, MODEL_DIM)
    mask_rows = memory_mask.reshape(ROWS).astype(jnp.float32)
    flag_rows = (
        jnp.broadcast_to(memory_collected_flag[:, None], (NUM_LABELS, MEM_PER_LABEL))
        .reshape(ROWS)
        .astype(jnp.float32)
    )
    mesh = plsc.VectorSubcoreMesh(core_axis_name="c", subcore_axis_name="s")
    k = functools.partial(
        pl.kernel,
        mesh=mesh,
        out_type=jax.ShapeDtypeStruct((ROWS, MODEL_DIM), jnp.float32),
        scratch_types=[
            pltpu.VMEM((NBUF, CHUNK, MODEL_DIM), jnp.float32),
            pltpu.VMEM((RPW,), jnp.float32),
            pltpu.VMEM((RPW,), jnp.float32),
            pltpu.SemaphoreType.DMA((NBUF,)),
            pltpu.SemaphoreType.DMA((NBUF,)),
            pltpu.SemaphoreType.DMA,
        ],
    )(_sc_body)
    return k(flat, mask_rows, flag_rows)
